# baseline (speedup 1.0000x reference)
import jax
import jax.numpy as jnp
from jax import lax
from jax.experimental import pallas as pl
from jax.experimental.pallas import tpu as pltpu
from jax.experimental.pallas import tpu_sc as plsc

BATCH = 16384
NC = 2
NS = 16
L = 16
NW = NC * NS
BPW = BATCH // NW
CHUNK = 128
NCH = BPW // CHUNK


def _irt_body(tab_hbm, uid_hbm, ida_hbm, idb_hbm, idc_hbm, out_hbm,
              uid_v, ida_v, idb_v, idc_v, th_v, a_v, b_v, c_v, out_v,
              idx_sem, out_sem, *chunk_sems):
    wid = lax.axis_index("s") * NC + lax.axis_index("c")
    base = wid * BPW

    bsl = pl.ds(base, BPW)
    ics = [pltpu.async_copy(uid_hbm.at[:, bsl], uid_v, idx_sem),
           pltpu.async_copy(ida_hbm.at[:, bsl], ida_v, idx_sem),
           pltpu.async_copy(idb_hbm.at[:, bsl], idb_v, idx_sem),
           pltpu.async_copy(idc_hbm.at[:, bsl], idc_v, idx_sem)]
    for ic in ics:
        ic.wait()

    copies = []
    for j in range(NCH):
        sl = pl.ds(j * CHUNK, CHUNK)
        sem = chunk_sems[j]
        copies.append((
            pltpu.async_copy(tab_hbm.at[uid_v.at[:, sl]], th_v.at[:, sl], sem),
            pltpu.async_copy(tab_hbm.at[ida_v.at[:, sl]], a_v.at[:, sl], sem),
            pltpu.async_copy(tab_hbm.at[idb_v.at[:, sl]], b_v.at[:, sl], sem),
            pltpu.async_copy(tab_hbm.at[idc_v.at[:, sl]], c_v.at[:, sl], sem),
        ))

    outs = []
    for j in range(NCH):
        for c in copies[j]:
            c.wait()
        for i in range(j * (CHUNK // L), (j + 1) * (CHUNK // L)):
            sl = pl.ds(i * L, L)
            th = th_v[0, sl]
            a = a_v[0, sl]
            b = b_v[0, sl]
            c = c_v[0, sl]
            cs = 1.0 / (1.0 + jnp.exp(-c))
            s = 1.0 / (1.0 + jnp.exp(-1.702 * a * (th - b)))
            out_v[sl] = cs + (1.0 - cs) * s
        outs.append(pltpu.async_copy(
            out_v.at[pl.ds(j * CHUNK, CHUNK)],
            out_hbm.at[pl.ds(base + j * CHUNK, CHUNK)], out_sem))
    for o in outs:
        o.wait()


@jax.jit
def _irt_sc(tab, uid, ida, idb, idc):
    mesh = plsc.VectorSubcoreMesh(core_axis_name="c", subcore_axis_name="s")
    return pl.kernel(
        _irt_body,
        mesh=mesh,
        compiler_params=pltpu.CompilerParams(use_tc_tiling_on_sc=False),
        out_type=jax.ShapeDtypeStruct((BATCH,), jnp.float32),
        scratch_types=[
            pltpu.VMEM((1, BPW), jnp.int32),
            pltpu.VMEM((1, BPW), jnp.int32),
            pltpu.VMEM((1, BPW), jnp.int32),
            pltpu.VMEM((1, BPW), jnp.int32),
            pltpu.VMEM((1, BPW), jnp.float32),
            pltpu.VMEM((1, BPW), jnp.float32),
            pltpu.VMEM((1, BPW), jnp.float32),
            pltpu.VMEM((1, BPW), jnp.float32),
            pltpu.VMEM((BPW,), jnp.float32),
            pltpu.SemaphoreType.DMA,
            pltpu.SemaphoreType.DMA,
        ] + [pltpu.SemaphoreType.DMA] * NCH,
    )(tab, uid, ida, idb, idc)


def kernel(user_id, item_id, theta_w, a_w, b_w, c_w):
    uid = user_id.astype(jnp.int32)[None, :]
    iid = item_id.astype(jnp.int32)
    tab = jnp.concatenate([theta_w, a_w, b_w, c_w], axis=0)
    tab = jnp.pad(tab, ((0, 480), (0, 0))).T
    iid_a = iid + 1000000
    iid_b = iid + 1100000
    iid_c = iid + 1200000
    return _irt_sc(tab, uid, iid_a[None, :], iid_b[None, :], iid_c[None, :])


# speedup vs baseline: 3.1365x; 3.1365x over previous
import jax
import jax.numpy as jnp
from jax import lax
from jax.experimental import pallas as pl
from jax.experimental.pallas import tpu as pltpu
from jax.experimental.pallas import tpu_sc as plsc

BATCH = 16384
NC = 2
NS = 16
L = 16
NW = NC * NS
BPW = BATCH // NW
CHUNK = 512
NCH = BPW // CHUNK


def _irt_body(theta_hbm, a_hbm, b_hbm, c_hbm, uid_hbm, iid_hbm, out_hbm,
              uid_v, iid_v, th_v, a_v, b_v, c_v, out_v,
              idx_sem, out_sem, *chunk_sems):
    wid = lax.axis_index("s") * NC + lax.axis_index("c")
    base = wid * BPW

    bsl = pl.ds(base, BPW)
    ic0 = pltpu.async_copy(uid_hbm.at[:, bsl], uid_v, idx_sem)
    ic1 = pltpu.async_copy(iid_hbm.at[:, bsl], iid_v, idx_sem)
    ic0.wait()
    ic1.wait()

    copies = []
    for j in range(NCH):
        sl = pl.ds(j * CHUNK, CHUNK)
        sem = chunk_sems[j]
        copies.append((
            pltpu.async_copy(theta_hbm.at[uid_v.at[:, sl]], th_v.at[:, sl], sem),
            pltpu.async_copy(a_hbm.at[iid_v.at[:, sl]], a_v.at[:, sl], sem),
            pltpu.async_copy(b_hbm.at[iid_v.at[:, sl]], b_v.at[:, sl], sem),
            pltpu.async_copy(c_hbm.at[iid_v.at[:, sl]], c_v.at[:, sl], sem),
        ))

    outs = []
    for j in range(NCH):
        for c in copies[j]:
            c.wait()
        for i in range(j * (CHUNK // L), (j + 1) * (CHUNK // L)):
            sl = pl.ds(i * L, L)
            th = th_v[0, sl]
            a = a_v[0, sl]
            b = b_v[0, sl]
            c = c_v[0, sl]
            cs = 1.0 / (1.0 + jnp.exp(-c))
            s = 1.0 / (1.0 + jnp.exp(-1.702 * a * (th - b)))
            out_v[sl] = cs + (1.0 - cs) * s
        outs.append(pltpu.async_copy(
            out_v.at[pl.ds(j * CHUNK, CHUNK)],
            out_hbm.at[pl.ds(base + j * CHUNK, CHUNK)], out_sem))
    for o in outs:
        o.wait()


@jax.jit
def _irt_sc(theta, a_tab, b_tab, c_tab, uid, iid):
    mesh = plsc.VectorSubcoreMesh(core_axis_name="c", subcore_axis_name="s")
    return pl.kernel(
        _irt_body,
        mesh=mesh,
        compiler_params=pltpu.CompilerParams(use_tc_tiling_on_sc=False),
        out_type=jax.ShapeDtypeStruct((BATCH,), jnp.float32),
        scratch_types=[
            pltpu.VMEM((1, BPW), jnp.int32),
            pltpu.VMEM((1, BPW), jnp.int32),
            pltpu.VMEM((1, BPW), jnp.float32),
            pltpu.VMEM((1, BPW), jnp.float32),
            pltpu.VMEM((1, BPW), jnp.float32),
            pltpu.VMEM((1, BPW), jnp.float32),
            pltpu.VMEM((BPW,), jnp.float32),
            pltpu.SemaphoreType.DMA,
            pltpu.SemaphoreType.DMA,
        ] + [pltpu.SemaphoreType.DMA] * NCH,
    )(theta, a_tab, b_tab, c_tab, uid, iid)


def kernel(user_id, item_id, theta_w, a_w, b_w, c_w):
    uid = user_id.astype(jnp.int32)[None, :]
    iid = item_id.astype(jnp.int32)[None, :]
    t = jnp.pad(theta_w, ((0, 448), (0, 0))).T
    a1 = jnp.pad(a_w, ((0, 352), (0, 0))).T
    b1 = jnp.pad(b_w, ((0, 352), (0, 0))).T
    c1 = jnp.pad(c_w, ((0, 352), (0, 0))).T
    return _irt_sc(t, a1, b1, c1, uid, iid)


# speedup vs baseline: 3.1568x; 1.0065x over previous
import jax
import jax.numpy as jnp
from jax import lax
from jax.experimental import pallas as pl
from jax.experimental.pallas import tpu as pltpu
from jax.experimental.pallas import tpu_sc as plsc

BATCH = 16384
NC = 2
NS = 16
L = 16
NW = NC * NS
BPW = BATCH // NW
CHUNK = 128
NCH = BPW // CHUNK


def _irt_body(theta_hbm, a_hbm, b_hbm, c_hbm, uid_hbm, iid_hbm, out_hbm,
              uid_v, iid_v, th_v, a_v, b_v, c_v, out_v,
              idx_sem, out_sem, *chunk_sems):
    wid = lax.axis_index("s") * NC + lax.axis_index("c")
    base = wid * BPW

    bsl = pl.ds(base, BPW)
    ic0 = pltpu.async_copy(uid_hbm.at[:, bsl], uid_v, idx_sem)
    ic1 = pltpu.async_copy(iid_hbm.at[:, bsl], iid_v, idx_sem)
    ic0.wait()
    ic1.wait()

    copies = []
    for j in range(NCH):
        sl = pl.ds(j * CHUNK, CHUNK)
        sem = chunk_sems[j]
        copies.append((
            pltpu.async_copy(theta_hbm.at[uid_v.at[:, sl]], th_v.at[:, sl], sem),
            pltpu.async_copy(a_hbm.at[iid_v.at[:, sl]], a_v.at[:, sl], sem),
            pltpu.async_copy(b_hbm.at[iid_v.at[:, sl]], b_v.at[:, sl], sem),
            pltpu.async_copy(c_hbm.at[iid_v.at[:, sl]], c_v.at[:, sl], sem),
        ))

    outs = []
    for j in range(NCH):
        for c in copies[j]:
            c.wait()
        for i in range(j * (CHUNK // L), (j + 1) * (CHUNK // L)):
            sl = pl.ds(i * L, L)
            th = th_v[0, sl]
            a = a_v[0, sl]
            b = b_v[0, sl]
            c = c_v[0, sl]
            cs = 1.0 / (1.0 + jnp.exp(-c))
            s = 1.0 / (1.0 + jnp.exp(-1.702 * a * (th - b)))
            out_v[sl] = cs + (1.0 - cs) * s
        outs.append(pltpu.async_copy(
            out_v.at[pl.ds(j * CHUNK, CHUNK)],
            out_hbm.at[pl.ds(base + j * CHUNK, CHUNK)], out_sem))
    for o in outs:
        o.wait()


@jax.jit
def _irt_sc(theta, a_tab, b_tab, c_tab, uid, iid):
    mesh = plsc.VectorSubcoreMesh(core_axis_name="c", subcore_axis_name="s")
    return pl.kernel(
        _irt_body,
        mesh=mesh,
        compiler_params=pltpu.CompilerParams(use_tc_tiling_on_sc=False),
        out_type=jax.ShapeDtypeStruct((BATCH,), jnp.float32),
        scratch_types=[
            pltpu.VMEM((1, BPW), jnp.int32),
            pltpu.VMEM((1, BPW), jnp.int32),
            pltpu.VMEM((1, BPW), jnp.float32),
            pltpu.VMEM((1, BPW), jnp.float32),
            pltpu.VMEM((1, BPW), jnp.float32),
            pltpu.VMEM((1, BPW), jnp.float32),
            pltpu.VMEM((BPW,), jnp.float32),
            pltpu.SemaphoreType.DMA,
            pltpu.SemaphoreType.DMA,
        ] + [pltpu.SemaphoreType.DMA] * NCH,
    )(theta, a_tab, b_tab, c_tab, uid, iid)


def kernel(user_id, item_id, theta_w, a_w, b_w, c_w):
    uid = user_id.astype(jnp.int32)[None, :]
    iid = item_id.astype(jnp.int32)[None, :]
    t = jnp.pad(theta_w, ((0, 448), (0, 0))).T
    a1 = jnp.pad(a_w, ((0, 352), (0, 0))).T
    b1 = jnp.pad(b_w, ((0, 352), (0, 0))).T
    c1 = jnp.pad(c_w, ((0, 352), (0, 0))).T
    return _irt_sc(t, a1, b1, c1, uid, iid)


# speedup vs baseline: 3.1574x; 1.0002x over previous
import jax
import jax.numpy as jnp
from jax import lax
from jax.experimental import pallas as pl
from jax.experimental.pallas import tpu as pltpu
from jax.experimental.pallas import tpu_sc as plsc

BATCH = 16384
NC = 2
NS = 16
L = 16
NW = NC * NS
BPW = BATCH // NW
CHUNK = 128
NCH = BPW // CHUNK


def _irt_body(theta_hbm, a_hbm, b_hbm, c_hbm, uid_hbm, iid_hbm, out_hbm,
              uid_v, iid_v, th_v, a_v, b_v, c_v, out_v,
              idx_sem, out_sem, *chunk_sems):
    wid = lax.axis_index("s") * NC + lax.axis_index("c")
    base = wid * BPW

    bsl = pl.ds(base, BPW)
    ic0 = pltpu.async_copy(uid_hbm.at[:, bsl], uid_v, idx_sem)
    ic1 = pltpu.async_copy(iid_hbm.at[:, bsl], iid_v, idx_sem)
    ic0.wait()
    ic1.wait()

    copies = []
    for j in range(NCH):
        sl = pl.ds(j * CHUNK, CHUNK)
        sem = chunk_sems[j]
        copies.append((
            pltpu.async_copy(theta_hbm.at[uid_v.at[:, sl]], th_v.at[:, sl], sem),
            pltpu.async_copy(a_hbm.at[iid_v.at[:, sl]], a_v.at[:, sl], sem),
            pltpu.async_copy(b_hbm.at[iid_v.at[:, sl]], b_v.at[:, sl], sem),
            pltpu.async_copy(c_hbm.at[iid_v.at[:, sl]], c_v.at[:, sl], sem),
        ))

    outs = []
    for j in range(NCH):
        for c in copies[j]:
            c.wait()
        for i in range(j * (CHUNK // L), (j + 1) * (CHUNK // L)):
            sl = pl.ds(i * L, L)
            th = th_v[0, sl]
            a = a_v[0, sl]
            b = b_v[0, sl]
            c = c_v[0, sl]
            cs = 1.0 / (1.0 + jnp.exp(-c))
            s = 1.0 / (1.0 + jnp.exp(-1.702 * a * (th - b)))
            out_v[sl] = cs + (1.0 - cs) * s
        outs.append(pltpu.async_copy(
            out_v.at[pl.ds(j * CHUNK, CHUNK)],
            out_hbm.at[pl.ds(base + j * CHUNK, CHUNK)], out_sem))
    for o in outs:
        o.wait()


@jax.jit
def _irt_sc(theta, a_tab, b_tab, c_tab, uid, iid):
    mesh = plsc.VectorSubcoreMesh(core_axis_name="c", subcore_axis_name="s")
    return pl.kernel(
        _irt_body,
        mesh=mesh,
        compiler_params=pltpu.CompilerParams(use_tc_tiling_on_sc=False),
        out_type=jax.ShapeDtypeStruct((BATCH,), jnp.float32),
        scratch_types=[
            pltpu.VMEM((1, BPW), jnp.int32),
            pltpu.VMEM((1, BPW), jnp.int32),
            pltpu.VMEM((1, BPW), jnp.float32),
            pltpu.VMEM((1, BPW), jnp.float32),
            pltpu.VMEM((1, BPW), jnp.float32),
            pltpu.VMEM((1, BPW), jnp.float32),
            pltpu.VMEM((BPW,), jnp.float32),
            pltpu.SemaphoreType.DMA,
            pltpu.SemaphoreType.DMA,
        ] + [pltpu.SemaphoreType.DMA] * NCH,
    )(theta, a_tab, b_tab, c_tab, uid, iid)


def kernel(user_id, item_id, theta_w, a_w, b_w, c_w):
    uid = user_id.astype(jnp.int32)[None, :]
    iid = item_id.astype(jnp.int32)[None, :]
    t = jnp.pad(theta_w.T, ((0, 0), (0, 448)))
    a1 = jnp.pad(a_w.T, ((0, 0), (0, 352)))
    b1 = jnp.pad(b_w.T, ((0, 0), (0, 352)))
    c1 = jnp.pad(c_w.T, ((0, 0), (0, 352)))
    return _irt_sc(t, a1, b1, c1, uid, iid)
